# baseline (speedup 1.0000x reference)
import functools

import jax
import jax.numpy as jnp
from jax import lax
from jax.experimental import pallas as pl
from jax.experimental.pallas import tpu as pltpu
from jax.experimental.pallas import tpu_sc as plsc

_B = 4096
_D = 256
_K = 8192
_IN = 1024
_BT = 512
_CT = 512
_I32_MAX = 2147483647


def _enc_vq_body(x_ref, w1, b1, w2, b2, w3, b3, w4, b4, emb_ref,
                 lat_ref, idx_ref):
    h = jnp.maximum(jnp.dot(x_ref[...], w1[...]) + b1[...], 0.0)
    h = jnp.maximum(jnp.dot(h, w2[...]) + b2[...], 0.0)
    h = jnp.maximum(jnp.dot(h, w3[...]) + b3[...], 0.0)
    lat = jnp.dot(h, w4[...]) + b4[...]
    lat_ref[...] = lat

    l2 = jnp.sum(lat * lat, axis=1, keepdims=True)
    ones_row = jnp.ones((1, _D), jnp.float32)

    lat2 = lat + lat
    cols = lax.broadcasted_iota(jnp.int32, (_BT, _CT), 1)

    def tile_step(j, carry):
        run_min, run_idx = carry
        start = pl.multiple_of(j * _CT, _CT)
        et = emb_ref[pl.ds(start, _CT), :]
        cl2 = lax.dot_general(lat2, et, (((1,), (1,)), ((), ())))
        c2 = lax.dot_general(ones_row, et * et, (((1,), (1,)), ((), ())))
        t = l2 - cl2
        s = t + c2
        m = jnp.min(s, axis=1, keepdims=True)
        lidx = jnp.min(jnp.where(s == m, cols, _I32_MAX),
                       axis=1, keepdims=True)
        gidx = lidx + j * _CT
        upd = m < run_min
        return (jnp.where(upd, m, run_min), jnp.where(upd, gidx, run_idx))

    init = (jnp.full((_BT, 1), jnp.inf, jnp.float32),
            jnp.zeros((_BT, 1), jnp.int32))
    _, best_idx = lax.fori_loop(0, _K // _CT, tile_step, init)
    idx_ref[...] = best_idx


def _encode_argmin(x2, w1, b1, w2, b2, w3, b3, w4, b4, emb):
    nb = _B // _BT
    full = lambda shape: pl.BlockSpec(shape, lambda i: (0,) * len(shape))
    return pl.pallas_call(
        _enc_vq_body,
        grid=(nb,),
        in_specs=[
            pl.BlockSpec((_BT, _IN), lambda i: (i, 0)),
            full((_IN, 128)), full((1, 128)),
            full((128, 64)), full((1, 64)),
            full((64, 32)), full((1, 32)),
            full((32, _D)), full((1, _D)),
            full((_K, _D)),
        ],
        out_specs=[
            pl.BlockSpec((_BT, _D), lambda i: (i, 0)),
            pl.BlockSpec((_BT, 1), lambda i: (i, 0)),
        ],
        out_shape=[
            jax.ShapeDtypeStruct((_B, _D), jnp.float32),
            jax.ShapeDtypeStruct((_B, 1), jnp.int32),
        ],
    )(x2, w1, b1, w2, b2, w3, b3, w4, b4, emb)


_NC = 2
_NS = 16
_NW = _NC * _NS
_BPW = _B // _NW


def _gather_codes_sc(emb, closest):
    mesh = plsc.VectorSubcoreMesh(core_axis_name="c", subcore_axis_name="s")

    @functools.partial(
        pl.kernel, mesh=mesh,
        out_type=jax.ShapeDtypeStruct((_B, _D), jnp.float32),
        scratch_types=[
            pltpu.VMEM((_BPW,), jnp.int32),
            pltpu.VMEM((_BPW, _D), jnp.float32),
            pltpu.SemaphoreType.DMA,
        ],
    )
    def gather(emb_hbm, idx_hbm, out_hbm, idx_v, rows_v, sem):
        wid = lax.axis_index("s") * _NC + lax.axis_index("c")
        base = wid * _BPW
        pltpu.sync_copy(idx_hbm.at[pl.ds(base, _BPW)], idx_v)
        pltpu.async_copy(emb_hbm.at[idx_v], rows_v, sem).wait()
        pltpu.sync_copy(rows_v, out_hbm.at[pl.ds(base, _BPW)])

    return gather(emb, closest)


def _dec_body(idx_sref, lat_ref, emb_ref, w5, b5, w6, b6, w7, b7, w8, b8,
              cs_ref, dec_ref, loss_ref, codes_ref):
    i = pl.program_id(0)
    base = i * _BT

    def gather_row(r, _):
        k = idx_sref[base + r]
        codes_ref[pl.ds(r, 1), :] = emb_ref[pl.ds(k, 1), :]
        return 0

    lax.fori_loop(0, _BT, gather_row, 0)

    lat = lat_ref[...]
    codes = codes_ref[...]
    diff = codes - lat
    cs = lat + diff
    cs_ref[...] = cs

    @pl.when(i == 0)
    def _():
        loss_ref[...] = jnp.zeros_like(loss_ref)

    part = jnp.sum(diff * diff)
    loss_ref[...] = loss_ref[...] + jnp.reshape(part, (1, 1))

    d = jnp.maximum(jnp.dot(cs, w5[...]) + b5[...], 0.0)
    d = jnp.maximum(jnp.dot(d, w6[...]) + b6[...], 0.0)
    d = jnp.maximum(jnp.dot(d, w7[...]) + b7[...], 0.0)
    dec_ref[...] = jax.nn.sigmoid(jnp.dot(d, w8[...]) + b8[...])


def _decode(closest, lat, emb, w5, b5, w6, b6, w7, b7, w8, b8):
    nb = _B // _BT
    full = lambda shape: pl.BlockSpec(shape, lambda i, idx: (0,) * len(shape))
    grid_spec = pltpu.PrefetchScalarGridSpec(
        num_scalar_prefetch=1,
        grid=(nb,),
        in_specs=[
            pl.BlockSpec((_BT, _D), lambda i, idx: (i, 0)),
            full((_K, _D)),
            full((_D, 32)), full((1, 32)),
            full((32, 64)), full((1, 64)),
            full((64, 128)), full((1, 128)),
            full((128, _IN)), full((1, _IN)),
        ],
        out_specs=[
            pl.BlockSpec((_BT, _D), lambda i, idx: (i, 0)),
            pl.BlockSpec((_BT, _IN), lambda i, idx: (i, 0)),
            pl.BlockSpec((1, 1), lambda i, idx: (0, 0)),
        ],
        scratch_shapes=[pltpu.VMEM((_BT, _D), jnp.float32)],
    )
    return pl.pallas_call(
        _dec_body,
        grid_spec=grid_spec,
        out_shape=[
            jax.ShapeDtypeStruct((_B, _D), jnp.float32),
            jax.ShapeDtypeStruct((_B, _IN), jnp.float32),
            jax.ShapeDtypeStruct((1, 1), jnp.float32),
        ],
    )(closest, lat, emb, w5, b5, w6, b6, w7, b7, w8, b8)


def kernel(x, W1, b1, W2, b2, W3, b3, W4, b4, emb,
           W5, b5, W6, b6, W7, b7, W8, b8):
    batch, channels, height, width = x.shape
    x2 = x.reshape(batch, -1)
    row = lambda b: b.reshape(1, -1)

    lat, closest2d = _encode_argmin(
        x2, W1, row(b1), W2, row(b2), W3, row(b3), W4, row(b4), emb)
    cs, dec, loss_sum = _decode(
        closest2d.reshape(_B), lat, emb,
        W5, row(b5), W6, row(b6), W7, row(b7), W8, row(b8))

    loss = loss_sum[0, 0] / (_B * _D)
    decoded = dec.reshape(batch, channels, height, width)
    return (lat, cs, decoded, loss, loss)


# speedup vs baseline: 1.2325x; 1.2325x over previous
import functools

import jax
import jax.numpy as jnp
from jax import lax
from jax.experimental import pallas as pl
from jax.experimental.pallas import tpu as pltpu
from jax.experimental.pallas import tpu_sc as plsc

_B = 4096
_D = 256
_K = 8192
_IN = 1024
_BT = 512
_CT = 2048
_I32_MAX = 2147483647


def _enc_vq_body(x_ref, w1, b1, w2, b2, w3, b3, w4, b4, emb_ref,
                 lat_ref, idx_ref):
    h = jnp.maximum(jnp.dot(x_ref[...], w1[...]) + b1[...], 0.0)
    h = jnp.maximum(jnp.dot(h, w2[...]) + b2[...], 0.0)
    h = jnp.maximum(jnp.dot(h, w3[...]) + b3[...], 0.0)
    lat = jnp.dot(h, w4[...]) + b4[...]
    lat_ref[...] = lat

    l2 = jnp.sum(lat * lat, axis=1, keepdims=True)
    ones_row = jnp.ones((1, _D), jnp.float32)

    lat2 = lat + lat
    cols = lax.broadcasted_iota(jnp.int32, (_BT, _CT), 1)

    def tile_step(j, carry):
        run_min, run_idx = carry
        start = pl.multiple_of(j * _CT, _CT)
        et = emb_ref[pl.ds(start, _CT), :]
        cl2 = lax.dot_general(lat2, et, (((1,), (1,)), ((), ())))
        c2 = lax.dot_general(ones_row, et * et, (((1,), (1,)), ((), ())))
        t = l2 - cl2
        s = t + c2
        m = jnp.min(s, axis=1, keepdims=True)
        lidx = jnp.min(jnp.where(s == m, cols, _I32_MAX),
                       axis=1, keepdims=True)
        gidx = lidx + j * _CT
        upd = m < run_min
        return (jnp.where(upd, m, run_min), jnp.where(upd, gidx, run_idx))

    init = (jnp.full((_BT, 1), jnp.inf, jnp.float32),
            jnp.zeros((_BT, 1), jnp.int32))
    _, best_idx = lax.fori_loop(0, _K // _CT, tile_step, init)
    idx_ref[...] = best_idx


def _encode_argmin(x2, w1, b1, w2, b2, w3, b3, w4, b4, emb):
    nb = _B // _BT
    full = lambda shape: pl.BlockSpec(shape, lambda i: (0,) * len(shape))
    return pl.pallas_call(
        _enc_vq_body,
        grid=(nb,),
        in_specs=[
            pl.BlockSpec((_BT, _IN), lambda i: (i, 0)),
            full((_IN, 128)), full((1, 128)),
            full((128, 64)), full((1, 64)),
            full((64, 32)), full((1, 32)),
            full((32, _D)), full((1, _D)),
            full((_K, _D)),
        ],
        out_specs=[
            pl.BlockSpec((_BT, _D), lambda i: (i, 0)),
            pl.BlockSpec((_BT, 1), lambda i: (i, 0)),
        ],
        out_shape=[
            jax.ShapeDtypeStruct((_B, _D), jnp.float32),
            jax.ShapeDtypeStruct((_B, 1), jnp.int32),
        ],
    )(x2, w1, b1, w2, b2, w3, b3, w4, b4, emb)


_NC = 2
_NS = 16
_NW = _NC * _NS
_BPW = _B // _NW


def _gather_codes_sc(emb, closest):
    mesh = plsc.VectorSubcoreMesh(core_axis_name="c", subcore_axis_name="s")

    @functools.partial(
        pl.kernel, mesh=mesh,
        out_type=jax.ShapeDtypeStruct((_B, _D), jnp.float32),
        scratch_types=[
            pltpu.VMEM((_BPW,), jnp.int32),
            pltpu.VMEM((_BPW, _D), jnp.float32),
            pltpu.SemaphoreType.DMA,
        ],
    )
    def gather(emb_hbm, idx_hbm, out_hbm, idx_v, rows_v, sem):
        wid = lax.axis_index("s") * _NC + lax.axis_index("c")
        base = wid * _BPW
        pltpu.sync_copy(idx_hbm.at[pl.ds(base, _BPW)], idx_v)
        pltpu.async_copy(emb_hbm.at[idx_v], rows_v, sem).wait()
        pltpu.sync_copy(rows_v, out_hbm.at[pl.ds(base, _BPW)])

    return gather(emb, closest)


def _dec_body(idx_sref, lat_ref, emb_ref, w5, b5, w6, b6, w7, b7, w8, b8,
              cs_ref, dec_ref, loss_ref, codes_ref):
    i = pl.program_id(0)
    base = i * _BT

    def gather_row(r, _):
        k = idx_sref[base + r]
        codes_ref[pl.ds(r, 1), :] = emb_ref[pl.ds(k, 1), :]
        return 0

    lax.fori_loop(0, _BT, gather_row, 0)

    lat = lat_ref[...]
    codes = codes_ref[...]
    diff = codes - lat
    cs = lat + diff
    cs_ref[...] = cs

    @pl.when(i == 0)
    def _():
        loss_ref[...] = jnp.zeros_like(loss_ref)

    part = jnp.sum(diff * diff)
    loss_ref[...] = loss_ref[...] + jnp.reshape(part, (1, 1))

    d = jnp.maximum(jnp.dot(cs, w5[...]) + b5[...], 0.0)
    d = jnp.maximum(jnp.dot(d, w6[...]) + b6[...], 0.0)
    d = jnp.maximum(jnp.dot(d, w7[...]) + b7[...], 0.0)
    dec_ref[...] = jax.nn.sigmoid(jnp.dot(d, w8[...]) + b8[...])


def _decode(closest, lat, emb, w5, b5, w6, b6, w7, b7, w8, b8):
    nb = _B // _BT
    full = lambda shape: pl.BlockSpec(shape, lambda i, idx: (0,) * len(shape))
    grid_spec = pltpu.PrefetchScalarGridSpec(
        num_scalar_prefetch=1,
        grid=(nb,),
        in_specs=[
            pl.BlockSpec((_BT, _D), lambda i, idx: (i, 0)),
            full((_K, _D)),
            full((_D, 32)), full((1, 32)),
            full((32, 64)), full((1, 64)),
            full((64, 128)), full((1, 128)),
            full((128, _IN)), full((1, _IN)),
        ],
        out_specs=[
            pl.BlockSpec((_BT, _D), lambda i, idx: (i, 0)),
            pl.BlockSpec((_BT, _IN), lambda i, idx: (i, 0)),
            pl.BlockSpec((1, 1), lambda i, idx: (0, 0)),
        ],
        scratch_shapes=[pltpu.VMEM((_BT, _D), jnp.float32)],
    )
    return pl.pallas_call(
        _dec_body,
        grid_spec=grid_spec,
        out_shape=[
            jax.ShapeDtypeStruct((_B, _D), jnp.float32),
            jax.ShapeDtypeStruct((_B, _IN), jnp.float32),
            jax.ShapeDtypeStruct((1, 1), jnp.float32),
        ],
    )(closest, lat, emb, w5, b5, w6, b6, w7, b7, w8, b8)


def kernel(x, W1, b1, W2, b2, W3, b3, W4, b4, emb,
           W5, b5, W6, b6, W7, b7, W8, b8):
    batch, channels, height, width = x.shape
    x2 = x.reshape(batch, -1)
    row = lambda b: b.reshape(1, -1)

    lat, closest2d = _encode_argmin(
        x2, W1, row(b1), W2, row(b2), W3, row(b3), W4, row(b4), emb)
    cs, dec, loss_sum = _decode(
        closest2d.reshape(_B), lat, emb,
        W5, row(b5), W6, row(b6), W7, row(b7), W8, row(b8))

    loss = loss_sum[0, 0] / (_B * _D)
    decoded = dec.reshape(batch, channels, height, width)
    return (lat, cs, decoded, loss, loss)


# speedup vs baseline: 1.3155x; 1.0673x over previous
import functools

import jax
import jax.numpy as jnp
from jax import lax
from jax.experimental import pallas as pl
from jax.experimental.pallas import tpu as pltpu
from jax.experimental.pallas import tpu_sc as plsc

_B = 4096
_D = 256
_K = 8192
_IN = 1024
_BT = 1024
_CT = 2048
_I32_MAX = 2147483647


def _enc_vq_body(x_ref, w1, b1, w2, b2, w3, b3, w4, b4, emb_ref,
                 lat_ref, idx_ref):
    h = jnp.maximum(jnp.dot(x_ref[...], w1[...]) + b1[...], 0.0)
    h = jnp.maximum(jnp.dot(h, w2[...]) + b2[...], 0.0)
    h = jnp.maximum(jnp.dot(h, w3[...]) + b3[...], 0.0)
    lat = jnp.dot(h, w4[...]) + b4[...]
    lat_ref[...] = lat

    l2 = jnp.sum(lat * lat, axis=1, keepdims=True)
    ones_row = jnp.ones((1, _D), jnp.float32)

    lat2 = lat + lat
    et = emb_ref[...]
    cl2 = lax.dot_general(lat2, et, (((1,), (1,)), ((), ())))
    c2 = lax.dot_general(ones_row, et * et, (((1,), (1,)), ((), ())))
    t = l2 - cl2
    s = t + c2
    m = jnp.min(s, axis=1, keepdims=True)
    cols = lax.broadcasted_iota(jnp.int32, (_BT, _K), 1)
    best_idx = jnp.min(jnp.where(s == m, cols, _I32_MAX),
                       axis=1, keepdims=True)
    idx_ref[...] = best_idx


def _encode_argmin(x2, w1, b1, w2, b2, w3, b3, w4, b4, emb):
    nb = _B // _BT
    full = lambda shape: pl.BlockSpec(shape, lambda i: (0,) * len(shape))
    return pl.pallas_call(
        _enc_vq_body,
        grid=(nb,),
        in_specs=[
            pl.BlockSpec((_BT, _IN), lambda i: (i, 0)),
            full((_IN, 128)), full((1, 128)),
            full((128, 64)), full((1, 64)),
            full((64, 32)), full((1, 32)),
            full((32, _D)), full((1, _D)),
            full((_K, _D)),
        ],
        out_specs=[
            pl.BlockSpec((_BT, _D), lambda i: (i, 0)),
            pl.BlockSpec((_BT, 1), lambda i: (i, 0)),
        ],
        out_shape=[
            jax.ShapeDtypeStruct((_B, _D), jnp.float32),
            jax.ShapeDtypeStruct((_B, 1), jnp.int32),
        ],
    )(x2, w1, b1, w2, b2, w3, b3, w4, b4, emb)


_NC = 2
_NS = 16
_NW = _NC * _NS
_BPW = _B // _NW


def _gather_codes_sc(emb, closest):
    mesh = plsc.VectorSubcoreMesh(core_axis_name="c", subcore_axis_name="s")

    @functools.partial(
        pl.kernel, mesh=mesh,
        out_type=jax.ShapeDtypeStruct((_B, _D), jnp.float32),
        scratch_types=[
            pltpu.VMEM((_BPW,), jnp.int32),
            pltpu.VMEM((_BPW, _D), jnp.float32),
            pltpu.SemaphoreType.DMA,
        ],
    )
    def gather(emb_hbm, idx_hbm, out_hbm, idx_v, rows_v, sem):
        wid = lax.axis_index("s") * _NC + lax.axis_index("c")
        base = wid * _BPW
        pltpu.sync_copy(idx_hbm.at[pl.ds(base, _BPW)], idx_v)
        pltpu.async_copy(emb_hbm.at[idx_v], rows_v, sem).wait()
        pltpu.sync_copy(rows_v, out_hbm.at[pl.ds(base, _BPW)])

    return gather(emb, closest)


def _dec_body(idx_sref, lat_ref, emb_ref, w5, b5, w6, b6, w7, b7, w8, b8,
              cs_ref, dec_ref, loss_ref, codes_ref):
    i = pl.program_id(0)
    base = i * _BT

    def gather_row(r, _):
        k = idx_sref[base + r]
        codes_ref[pl.ds(r, 1), :] = emb_ref[pl.ds(k, 1), :]
        return 0

    lax.fori_loop(0, _BT, gather_row, 0)

    lat = lat_ref[...]
    codes = codes_ref[...]
    diff = codes - lat
    cs = lat + diff
    cs_ref[...] = cs

    @pl.when(i == 0)
    def _():
        loss_ref[...] = jnp.zeros_like(loss_ref)

    part = jnp.sum(diff * diff)
    loss_ref[...] = loss_ref[...] + jnp.reshape(part, (1, 1))

    d = jnp.maximum(jnp.dot(cs, w5[...]) + b5[...], 0.0)
    d = jnp.maximum(jnp.dot(d, w6[...]) + b6[...], 0.0)
    d = jnp.maximum(jnp.dot(d, w7[...]) + b7[...], 0.0)
    dec_ref[...] = jax.nn.sigmoid(jnp.dot(d, w8[...]) + b8[...])


def _decode(closest, lat, emb, w5, b5, w6, b6, w7, b7, w8, b8):
    nb = _B // _BT
    full = lambda shape: pl.BlockSpec(shape, lambda i, idx: (0,) * len(shape))
    grid_spec = pltpu.PrefetchScalarGridSpec(
        num_scalar_prefetch=1,
        grid=(nb,),
        in_specs=[
            pl.BlockSpec((_BT, _D), lambda i, idx: (i, 0)),
            full((_K, _D)),
            full((_D, 32)), full((1, 32)),
            full((32, 64)), full((1, 64)),
            full((64, 128)), full((1, 128)),
            full((128, _IN)), full((1, _IN)),
        ],
        out_specs=[
            pl.BlockSpec((_BT, _D), lambda i, idx: (i, 0)),
            pl.BlockSpec((_BT, _IN), lambda i, idx: (i, 0)),
            pl.BlockSpec((1, 1), lambda i, idx: (0, 0)),
        ],
        scratch_shapes=[pltpu.VMEM((_BT, _D), jnp.float32)],
    )
    return pl.pallas_call(
        _dec_body,
        grid_spec=grid_spec,
        out_shape=[
            jax.ShapeDtypeStruct((_B, _D), jnp.float32),
            jax.ShapeDtypeStruct((_B, _IN), jnp.float32),
            jax.ShapeDtypeStruct((1, 1), jnp.float32),
        ],
    )(closest, lat, emb, w5, b5, w6, b6, w7, b7, w8, b8)


def kernel(x, W1, b1, W2, b2, W3, b3, W4, b4, emb,
           W5, b5, W6, b6, W7, b7, W8, b8):
    batch, channels, height, width = x.shape
    x2 = x.reshape(batch, -1)
    row = lambda b: b.reshape(1, -1)

    lat, closest2d = _encode_argmin(
        x2, W1, row(b1), W2, row(b2), W3, row(b3), W4, row(b4), emb)
    cs, dec, loss_sum = _decode(
        closest2d.reshape(_B), lat, emb,
        W5, row(b5), W6, row(b6), W7, row(b7), W8, row(b8))

    loss = loss_sum[0, 0] / (_B * _D)
    decoded = dec.reshape(batch, channels, height, width)
    return (lat, cs, decoded, loss, loss)


# speedup vs baseline: 1.4203x; 1.0797x over previous
import functools

import jax
import jax.numpy as jnp
from jax import lax
from jax.experimental import pallas as pl
from jax.experimental.pallas import tpu as pltpu
from jax.experimental.pallas import tpu_sc as plsc

_B = 4096
_D = 256
_K = 8192
_IN = 1024
_BT = 1024
_CT = 2048
_I32_MAX = 2147483647


def _enc_vq_body(x_ref, w1, b1, w2, b2, w3, b3, w4, b4, emb_ref,
                 lat_ref, idx_ref):
    h = jnp.maximum(jnp.dot(x_ref[...], w1[...]) + b1[...], 0.0)
    h = jnp.maximum(jnp.dot(h, w2[...]) + b2[...], 0.0)
    h = jnp.maximum(jnp.dot(h, w3[...]) + b3[...], 0.0)
    lat = jnp.dot(h, w4[...]) + b4[...]
    lat_ref[...] = lat

    l2 = jnp.sum(lat * lat, axis=1, keepdims=True)
    ones_row = jnp.ones((1, _D), jnp.float32)

    lat2 = lat + lat
    et = emb_ref[...]
    cl2 = lax.dot_general(lat2, et, (((1,), (1,)), ((), ())))
    c2 = lax.dot_general(ones_row, et * et, (((1,), (1,)), ((), ())))
    t = l2 - cl2
    s = t + c2
    m = jnp.min(s, axis=1, keepdims=True)
    cols = lax.broadcasted_iota(jnp.int32, (_BT, _K), 1)
    best_idx = jnp.min(jnp.where(s == m, cols, _I32_MAX),
                       axis=1, keepdims=True)
    idx_ref[...] = best_idx


def _encode_argmin(x2, w1, b1, w2, b2, w3, b3, w4, b4, emb):
    nb = _B // _BT
    full = lambda shape: pl.BlockSpec(shape, lambda i: (0,) * len(shape))
    return pl.pallas_call(
        _enc_vq_body,
        grid=(nb,),
        in_specs=[
            pl.BlockSpec((_BT, _IN), lambda i: (i, 0)),
            full((_IN, 128)), full((1, 128)),
            full((128, 64)), full((1, 64)),
            full((64, 32)), full((1, 32)),
            full((32, _D)), full((1, _D)),
            full((_K, _D)),
        ],
        out_specs=[
            pl.BlockSpec((_BT, _D), lambda i: (i, 0)),
            pl.BlockSpec((_BT, 1), lambda i: (i, 0)),
        ],
        out_shape=[
            jax.ShapeDtypeStruct((_B, _D), jnp.float32),
            jax.ShapeDtypeStruct((_B, 1), jnp.int32),
        ],
    )(x2, w1, b1, w2, b2, w3, b3, w4, b4, emb)


_NC = 2
_NS = 16
_NW = _NC * _NS
_BPW = _B // _NW


def _gather_codes_sc(emb, closest):
    mesh = plsc.VectorSubcoreMesh(core_axis_name="c", subcore_axis_name="s")

    @functools.partial(
        pl.kernel, mesh=mesh,
        out_type=jax.ShapeDtypeStruct((_B, _D), jnp.float32),
        scratch_types=[
            pltpu.VMEM((_BPW,), jnp.int32),
            pltpu.VMEM((_BPW, _D), jnp.float32),
            pltpu.SemaphoreType.DMA,
        ],
    )
    def gather(emb_hbm, idx_hbm, out_hbm, idx_v, rows_v, sem):
        wid = lax.axis_index("s") * _NC + lax.axis_index("c")
        base = wid * _BPW
        pltpu.sync_copy(idx_hbm.at[pl.ds(base, _BPW)], idx_v)
        pltpu.async_copy(emb_hbm.at[idx_v], rows_v, sem).wait()
        pltpu.sync_copy(rows_v, out_hbm.at[pl.ds(base, _BPW)])

    return gather(emb, closest)


def _dec_body(idx_sref, lat_ref, emb_ref, w5, b5, w6, b6, w7, b7, w8, b8,
              cs_ref, dec_ref, loss_ref, codes_ref):
    i = pl.program_id(0)
    base = i * _BT

    def gather_rows(g, _):
        r0 = g * 8
        for u in range(8):
            k = idx_sref[base + r0 + u]
            codes_ref[pl.ds(r0 + u, 1), :] = emb_ref[pl.ds(k, 1), :]
        return 0

    lax.fori_loop(0, _BT // 8, gather_rows, 0)

    lat = lat_ref[...]
    codes = codes_ref[...]
    diff = codes - lat
    cs = lat + diff
    cs_ref[...] = cs

    @pl.when(i == 0)
    def _():
        loss_ref[...] = jnp.zeros_like(loss_ref)

    part = jnp.sum(diff * diff)
    loss_ref[...] = loss_ref[...] + jnp.reshape(part, (1, 1))

    d = jnp.maximum(jnp.dot(cs, w5[...]) + b5[...], 0.0)
    d = jnp.maximum(jnp.dot(d, w6[...]) + b6[...], 0.0)
    d = jnp.maximum(jnp.dot(d, w7[...]) + b7[...], 0.0)
    dec_ref[...] = jax.nn.sigmoid(jnp.dot(d, w8[...]) + b8[...])


def _decode(closest, lat, emb, w5, b5, w6, b6, w7, b7, w8, b8):
    nb = _B // _BT
    full = lambda shape: pl.BlockSpec(shape, lambda i, idx: (0,) * len(shape))
    grid_spec = pltpu.PrefetchScalarGridSpec(
        num_scalar_prefetch=1,
        grid=(nb,),
        in_specs=[
            pl.BlockSpec((_BT, _D), lambda i, idx: (i, 0)),
            full((_K, _D)),
            full((_D, 32)), full((1, 32)),
            full((32, 64)), full((1, 64)),
            full((64, 128)), full((1, 128)),
            full((128, _IN)), full((1, _IN)),
        ],
        out_specs=[
            pl.BlockSpec((_BT, _D), lambda i, idx: (i, 0)),
            pl.BlockSpec((_BT, _IN), lambda i, idx: (i, 0)),
            pl.BlockSpec((1, 1), lambda i, idx: (0, 0)),
        ],
        scratch_shapes=[pltpu.VMEM((_BT, _D), jnp.float32)],
    )
    return pl.pallas_call(
        _dec_body,
        grid_spec=grid_spec,
        out_shape=[
            jax.ShapeDtypeStruct((_B, _D), jnp.float32),
            jax.ShapeDtypeStruct((_B, _IN), jnp.float32),
            jax.ShapeDtypeStruct((1, 1), jnp.float32),
        ],
    )(closest, lat, emb, w5, b5, w6, b6, w7, b7, w8, b8)


def kernel(x, W1, b1, W2, b2, W3, b3, W4, b4, emb,
           W5, b5, W6, b6, W7, b7, W8, b8):
    batch, channels, height, width = x.shape
    x2 = x.reshape(batch, -1)
    row = lambda b: b.reshape(1, -1)

    lat, closest2d = _encode_argmin(
        x2, W1, row(b1), W2, row(b2), W3, row(b3), W4, row(b4), emb)
    cs, dec, loss_sum = _decode(
        closest2d.reshape(_B), lat, emb,
        W5, row(b5), W6, row(b6), W7, row(b7), W8, row(b8))

    loss = loss_sum[0, 0] / (_B * _D)
    decoded = dec.reshape(batch, channels, height, width)
    return (lat, cs, decoded, loss, loss)


# speedup vs baseline: 1.4207x; 1.0003x over previous
import functools

import jax
import jax.numpy as jnp
from jax import lax
from jax.experimental import pallas as pl
from jax.experimental.pallas import tpu as pltpu
from jax.experimental.pallas import tpu_sc as plsc

_B = 4096
_D = 256
_K = 8192
_IN = 1024
_BT = 1024
_I32_MAX = 2147483647


def _enc_vq_body(x_ref, w1, b1, w2, b2, w3, b3, w4, b4, emb_ref,
                 lat_ref, idx_ref):
    h = jnp.maximum(jnp.dot(x_ref[...], w1[...]) + b1[...], 0.0)
    h = jnp.maximum(jnp.dot(h, w2[...]) + b2[...], 0.0)
    h = jnp.maximum(jnp.dot(h, w3[...]) + b3[...], 0.0)
    lat = jnp.dot(h, w4[...]) + b4[...]
    lat_ref[...] = lat

    l2 = jnp.sum(lat * lat, axis=1, keepdims=True)
    ones_row = jnp.ones((1, _D), jnp.float32)

    lat2 = lat + lat
    et = emb_ref[...]
    cl2 = lax.dot_general(lat2, et, (((1,), (1,)), ((), ())))
    c2 = lax.dot_general(ones_row, et * et, (((1,), (1,)), ((), ())))
    t = l2 - cl2
    s = t + c2
    m = jnp.min(s, axis=1, keepdims=True)
    cols = lax.broadcasted_iota(jnp.int32, (_BT, _K), 1)
    best_idx = jnp.min(jnp.where(s == m, cols, _I32_MAX),
                       axis=1, keepdims=True)
    idx_ref[...] = best_idx


def _encode_argmin(x2, w1, b1, w2, b2, w3, b3, w4, b4, emb):
    nb = _B // _BT
    full = lambda shape: pl.BlockSpec(shape, lambda i: (0,) * len(shape))
    return pl.pallas_call(
        _enc_vq_body,
        grid=(nb,),
        in_specs=[
            pl.BlockSpec((_BT, _IN), lambda i: (i, 0)),
            full((_IN, 128)), full((1, 128)),
            full((128, 64)), full((1, 64)),
            full((64, 32)), full((1, 32)),
            full((32, _D)), full((1, _D)),
            full((_K, _D)),
        ],
        out_specs=[
            pl.BlockSpec((_BT, _D), lambda i: (i, 0)),
            pl.BlockSpec((_BT, 1), lambda i: (i, 0)),
        ],
        out_shape=[
            jax.ShapeDtypeStruct((_B, _D), jnp.float32),
            jax.ShapeDtypeStruct((_B, 1), jnp.int32),
        ],
    )(x2, w1, b1, w2, b2, w3, b3, w4, b4, emb)


_NC = 2
_NS = 16
_NW = _NC * _NS
_BPW = _B // _NW


def _gather_codes_sc(emb, closest):
    mesh = plsc.VectorSubcoreMesh(core_axis_name="c", subcore_axis_name="s")

    @functools.partial(
        pl.kernel, mesh=mesh,
        out_type=jax.ShapeDtypeStruct((_B, _D), jnp.float32),
        scratch_types=[
            pltpu.VMEM((_BPW,), jnp.int32),
            pltpu.VMEM((_BPW, _D), jnp.float32),
            pltpu.SemaphoreType.DMA,
        ],
    )
    def gather(emb_hbm, idx_hbm, out_hbm, idx_v, rows_v, sem):
        wid = lax.axis_index("s") * _NC + lax.axis_index("c")
        base = wid * _BPW
        pltpu.sync_copy(idx_hbm.at[pl.ds(base, _BPW)], idx_v)
        pltpu.async_copy(emb_hbm.at[idx_v], rows_v, sem).wait()
        pltpu.sync_copy(rows_v, out_hbm.at[pl.ds(base, _BPW)])

    return gather(emb, closest)


def _dec_body(idx_sref, lat_ref, emb_ref, w5, b5, w6, b6, w7, b7, w8, b8,
              cs_ref, dec_ref, loss_ref, codes_ref):
    i = pl.program_id(0)
    base = i * _BT

    def gather_rows(g, _):
        r0 = g * 8
        for u in range(8):
            k = idx_sref[base + r0 + u]
            codes_ref[pl.ds(r0 + u, 1), :] = emb_ref[pl.ds(k, 1), :]
        return 0

    lax.fori_loop(0, _BT // 8, gather_rows, 0)

    lat = lat_ref[...]
    codes = codes_ref[...]
    diff = codes - lat
    cs = lat + diff
    cs_ref[...] = cs

    @pl.when(i == 0)
    def _():
        loss_ref[...] = jnp.zeros_like(loss_ref)

    part = jnp.sum(diff * diff)
    loss_ref[...] = loss_ref[...] + jnp.reshape(part, (1, 1))

    d = jnp.maximum(jnp.dot(cs, w5[...]) + b5[...], 0.0)
    d = jnp.maximum(jnp.dot(d, w6[...]) + b6[...], 0.0)
    d = jnp.maximum(jnp.dot(d, w7[...]) + b7[...], 0.0)
    dec_ref[...] = jax.nn.sigmoid(jnp.dot(d, w8[...]) + b8[...])


def _decode(closest, lat, emb, w5, b5, w6, b6, w7, b7, w8, b8):
    nb = _B // _BT
    full = lambda shape: pl.BlockSpec(shape, lambda i, idx: (0,) * len(shape))
    grid_spec = pltpu.PrefetchScalarGridSpec(
        num_scalar_prefetch=1,
        grid=(nb,),
        in_specs=[
            pl.BlockSpec((_BT, _D), lambda i, idx: (i, 0)),
            full((_K, _D)),
            full((_D, 32)), full((1, 32)),
            full((32, 64)), full((1, 64)),
            full((64, 128)), full((1, 128)),
            full((128, _IN)), full((1, _IN)),
        ],
        out_specs=[
            pl.BlockSpec((_BT, _D), lambda i, idx: (i, 0)),
            pl.BlockSpec((_BT, _IN), lambda i, idx: (i, 0)),
            pl.BlockSpec((1, 1), lambda i, idx: (0, 0)),
        ],
        scratch_shapes=[pltpu.VMEM((_BT, _D), jnp.float32)],
    )
    return pl.pallas_call(
        _dec_body,
        grid_spec=grid_spec,
        out_shape=[
            jax.ShapeDtypeStruct((_B, _D), jnp.float32),
            jax.ShapeDtypeStruct((_B, _IN), jnp.float32),
            jax.ShapeDtypeStruct((1, 1), jnp.float32),
        ],
    )(closest, lat, emb, w5, b5, w6, b6, w7, b7, w8, b8)


def kernel(x, W1, b1, W2, b2, W3, b3, W4, b4, emb,
           W5, b5, W6, b6, W7, b7, W8, b8):
    batch, channels, height, width = x.shape
    x2 = x.reshape(batch, -1)
    row = lambda b: b.reshape(1, -1)

    lat, closest2d = _encode_argmin(
        x2, W1, row(b1), W2, row(b2), W3, row(b3), W4, row(b4), emb)
    cs, dec, loss_sum = _decode(
        closest2d.reshape(_B), lat, emb,
        W5, row(b5), W6, row(b6), W7, row(b7), W8, row(b8))

    loss = loss_sum[0, 0] / (_B * _D)
    decoded = dec.reshape(batch, channels, height, width)
    return (lat, cs, decoded, loss, loss)
